# baseline (speedup 1.0000x reference)
import jax
from jax.experimental import pallas as pl
from jax.experimental.pallas import tpu as pltpu


def _identity_copy(x_ref, o_ref, sem):
    cp = pltpu.make_async_copy(x_ref, o_ref, sem)
    cp.start()
    cp.wait()


def kernel(x, H, W):
    y = pl.pallas_call(
        _identity_copy,
        out_shape=jax.ShapeDtypeStruct(x.shape, x.dtype),
        in_specs=[pl.BlockSpec(memory_space=pl.ANY)],
        out_specs=pl.BlockSpec(memory_space=pl.ANY),
        scratch_shapes=[pltpu.SemaphoreType.DMA],
    )(x)
    return (y, H, W)


# speedup vs baseline: 26.0039x; 26.0039x over previous
import jax
from jax.experimental import pallas as pl
from jax.experimental.pallas import tpu as pltpu

_ROWS = 4 * 1024
_COLS = 768
_BLOCK_ROWS = 512


def _identity_copy(x_ref, o_ref):
    o_ref[...] = x_ref[...]


def kernel(x, H, W):
    x2 = x.reshape(_ROWS, _COLS)
    y = pl.pallas_call(
        _identity_copy,
        out_shape=jax.ShapeDtypeStruct((_ROWS, _COLS), x.dtype),
        grid=(_ROWS // _BLOCK_ROWS,),
        in_specs=[pl.BlockSpec((_BLOCK_ROWS, _COLS), lambda i: (i, 0))],
        out_specs=pl.BlockSpec((_BLOCK_ROWS, _COLS), lambda i: (i, 0)),
    )(x2)
    return (y.reshape(x.shape), H, W)


# speedup vs baseline: 27.9325x; 1.0742x over previous
import jax
from jax.experimental import pallas as pl
from jax.experimental.pallas import tpu as pltpu

_ROWS = 4 * 1024
_COLS = 768
_BLOCK_ROWS = 1024


def _identity_copy(x_ref, o_ref):
    o_ref[...] = x_ref[...]


def kernel(x, H, W):
    x2 = x.reshape(_ROWS, _COLS)
    y = pl.pallas_call(
        _identity_copy,
        out_shape=jax.ShapeDtypeStruct((_ROWS, _COLS), x.dtype),
        grid=(_ROWS // _BLOCK_ROWS,),
        in_specs=[pl.BlockSpec((_BLOCK_ROWS, _COLS), lambda i: (i, 0))],
        out_specs=pl.BlockSpec((_BLOCK_ROWS, _COLS), lambda i: (i, 0)),
    )(x2)
    return (y.reshape(x.shape), H, W)


# speedup vs baseline: 30.9323x; 1.1074x over previous
import jax
from jax.experimental import pallas as pl
from jax.experimental.pallas import tpu as pltpu

_ROWS = 4 * 1024
_COLS = 768
_BLOCK_ROWS = 2048


def _identity_copy(x_ref, o_ref):
    o_ref[...] = x_ref[...]


def kernel(x, H, W):
    x2 = x.reshape(_ROWS, _COLS)
    y = pl.pallas_call(
        _identity_copy,
        out_shape=jax.ShapeDtypeStruct((_ROWS, _COLS), x.dtype),
        grid=(_ROWS // _BLOCK_ROWS,),
        in_specs=[pl.BlockSpec((_BLOCK_ROWS, _COLS), lambda i: (i, 0))],
        out_specs=pl.BlockSpec((_BLOCK_ROWS, _COLS), lambda i: (i, 0)),
    )(x2)
    return (y.reshape(x.shape), H, W)


# speedup vs baseline: 31.8464x; 1.0296x over previous
import jax
from jax.experimental import pallas as pl
from jax.experimental.pallas import tpu as pltpu

_ROWS = 4 * 1024
_COLS = 768
_NCHUNK = 8
_CROWS = _ROWS // _NCHUNK


def _identity_copy(x_ref, o_ref, buf, in_sems, out_sems):
    for i in range(_NCHUNK):
        sl = pl.ds(i * _CROWS, _CROWS)
        pltpu.make_async_copy(x_ref.at[sl], buf.at[sl], in_sems.at[i]).start()
    for i in range(_NCHUNK):
        sl = pl.ds(i * _CROWS, _CROWS)
        pltpu.make_async_copy(x_ref.at[sl], buf.at[sl], in_sems.at[i]).wait()
        pltpu.make_async_copy(buf.at[sl], o_ref.at[sl], out_sems.at[i]).start()
    for i in range(_NCHUNK):
        sl = pl.ds(i * _CROWS, _CROWS)
        pltpu.make_async_copy(buf.at[sl], o_ref.at[sl], out_sems.at[i]).wait()


def kernel(x, H, W):
    x2 = x.reshape(_ROWS, _COLS)
    y = pl.pallas_call(
        _identity_copy,
        out_shape=jax.ShapeDtypeStruct((_ROWS, _COLS), x.dtype),
        in_specs=[pl.BlockSpec(memory_space=pl.ANY)],
        out_specs=pl.BlockSpec(memory_space=pl.ANY),
        scratch_shapes=[
            pltpu.VMEM((_ROWS, _COLS), x.dtype),
            pltpu.SemaphoreType.DMA((_NCHUNK,)),
            pltpu.SemaphoreType.DMA((_NCHUNK,)),
        ],
    )(x2)
    return (y.reshape(x.shape), H, W)
